# baseline (speedup 1.0000x reference)
import functools

import jax
import jax.numpy as jnp
from jax import lax
from jax.experimental import pallas as pl
from jax.experimental.pallas import tpu as pltpu
from jax.experimental.pallas import tpu_sc as plsc

N_PAD = 10240
NTILES = 32
CHUNK = 128
ROWS_PER_TILE = N_PAD // 16
BLK = 1024
S_HEAVY = 102
HEAVY_CORE = 0


def _sc_mesh():
    return plsc.VectorSubcoreMesh(core_axis_name="c", subcore_axis_name="s")


def _deg_body(dst_hbm, out_hbm, dst_v, buf, acc_sh, ssem):
    c = lax.axis_index("c")
    sid = lax.axis_index("s")
    wid = c * 16 + sid
    S = dst_v.shape[0]
    one16 = jnp.ones((16,), jnp.float32)
    zero16 = jnp.zeros((16,), jnp.float32)

    def zb(i, _):
        buf[i // 8, pl.ds((i % 8) * 16, 16)] = zero16
        return 0

    lax.fori_loop(0, CHUNK * 8, zb, 0)
    r0 = sid * ROWS_PER_TILE
    for j in range(ROWS_PER_TILE // CHUNK):
        pltpu.sync_copy(buf, acc_sh.at[pl.ds(r0 + j * CHUNK, CHUNK)])
    rem = ROWS_PER_TILE % CHUNK
    if rem:
        pltpu.sync_copy(
            buf.at[pl.ds(0, rem)],
            acc_sh.at[pl.ds(r0 + (ROWS_PER_TILE // CHUNK) * CHUNK, rem)])

    def ob(i, _):
        buf[i // 8, pl.ds((i % 8) * 16, 16)] = one16
        return 0

    lax.fori_loop(0, CHUNK * 8, ob, 0)
    pltpu.sync_copy(dst_hbm.at[wid], dst_v)
    plsc.subcore_barrier()

    def step(t, _):
        pltpu.async_copy(buf, acc_sh.at[dst_v.at[t]], ssem, add=True)
        return 0

    lax.fori_loop(0, S, step, 0)

    def drain(t, _):
        pltpu.make_async_copy(buf, acc_sh.at[dst_v.at[t]], ssem).wait()
        return 0

    lax.fori_loop(0, S, drain, 0)
    plsc.subcore_barrier()
    pltpu.sync_copy(acc_sh.at[pl.ds(r0, ROWS_PER_TILE)],
                    out_hbm.at[c, pl.ds(r0, ROWS_PER_TILE)])


def _sc_degree(dst3):
    S = dst3.shape[1]
    fn = pl.kernel(
        _deg_body,
        out_type=jax.ShapeDtypeStruct((2, N_PAD, 128), jnp.float32),
        mesh=_sc_mesh(),
        scratch_types=[
            pltpu.VMEM((S, CHUNK), jnp.int32),
            pltpu.VMEM((CHUNK, 128), jnp.float32),
            pltpu.VMEM_SHARED((N_PAD, 128), jnp.float32),
            pltpu.SemaphoreType.DMA,
        ],
    )
    return fn(dst3)


def _agg_body(p0, p1, h_hbm, src_hbm, dst_hbm, out_hbm, src_v, dstr, buf0,
              buf1, acc_sh, gs0, gs1, ss0, ss1, ds0, ds1):
    c = lax.axis_index("c")
    sid = lax.axis_index("s")
    wid = c * 16 + sid
    P = lax.convert_element_type(p0 + (p1 - p0) * c, jnp.int32)
    zero16 = jnp.zeros((16,), jnp.float32)

    def zb(i, _):
        buf0[i // 8, pl.ds((i % 8) * 16, 16)] = zero16
        return 0

    lax.fori_loop(0, CHUNK * 8, zb, 0)
    r0 = sid * ROWS_PER_TILE
    for j in range(ROWS_PER_TILE // CHUNK):
        pltpu.sync_copy(buf0, acc_sh.at[pl.ds(r0 + j * CHUNK, CHUNK)])
    pltpu.sync_copy(src_hbm.at[wid], src_v)
    plsc.subcore_barrier()

    pltpu.async_copy(dst_hbm.at[wid, 0], dstr.at[0], ds0)
    pltpu.async_copy(dst_hbm.at[wid, 1], dstr.at[1], ds1)
    pltpu.async_copy(h_hbm.at[src_v.at[0]], buf0, gs0)
    pltpu.async_copy(h_hbm.at[src_v.at[1]], buf1, gs1)

    def pair(p, _):
        t0 = 2 * p
        t1 = t0 + 1
        pltpu.make_async_copy(h_hbm.at[src_v.at[t0]], buf0, gs0).wait()
        pltpu.make_async_copy(dst_hbm.at[wid, t0], dstr.at[0], ds0).wait()
        pltpu.async_copy(buf0, acc_sh.at[dstr.at[0]], ss0, add=True)
        pltpu.make_async_copy(h_hbm.at[src_v.at[t1]], buf1, gs1).wait()
        pltpu.make_async_copy(dst_hbm.at[wid, t1], dstr.at[1], ds1).wait()
        pltpu.async_copy(buf1, acc_sh.at[dstr.at[1]], ss1, add=True)
        pltpu.make_async_copy(buf0, acc_sh.at[dstr.at[0]], ss0).wait()

        @pl.when(p + 1 < P)
        def _():
            pltpu.async_copy(dst_hbm.at[wid, t0 + 2], dstr.at[0], ds0)
            pltpu.async_copy(h_hbm.at[src_v.at[t0 + 2]], buf0, gs0)

        pltpu.make_async_copy(buf1, acc_sh.at[dstr.at[1]], ss1).wait()

        @pl.when(p + 1 < P)
        def _():
            pltpu.async_copy(dst_hbm.at[wid, t1 + 2], dstr.at[1], ds1)
            pltpu.async_copy(h_hbm.at[src_v.at[t1 + 2]], buf1, gs1)

        return 0

    lax.fori_loop(0, P, pair, 0)
    plsc.subcore_barrier()
    pltpu.sync_copy(acc_sh.at[pl.ds(r0, ROWS_PER_TILE)],
                    out_hbm.at[c, pl.ds(r0, ROWS_PER_TILE)])


def _sc_aggregate(h, src3, dst3, s0, s1):
    S = src3.shape[1]
    fn = pl.kernel(
        functools.partial(_agg_body, s0 // 2, s1 // 2),
        out_type=jax.ShapeDtypeStruct((2, N_PAD, 128), jnp.float32),
        mesh=_sc_mesh(),
        scratch_types=[
            pltpu.VMEM((S, CHUNK), jnp.int32),
            pltpu.VMEM((2, CHUNK), jnp.int32),
            pltpu.VMEM((CHUNK, 128), jnp.float32),
            pltpu.VMEM((CHUNK, 128), jnp.float32),
            pltpu.VMEM_SHARED((N_PAD, 128), jnp.float32),
            pltpu.SemaphoreType.DMA,
            pltpu.SemaphoreType.DMA,
            pltpu.SemaphoreType.DMA,
            pltpu.SemaphoreType.DMA,
            pltpu.SemaphoreType.DMA,
            pltpu.SemaphoreType.DMA,
        ],
    )
    return fn(h, src3, dst3)


def _dinv_of(dg_ref):
    deg = dg_ref[0, :, 0:1] + dg_ref[1, :, 0:1]
    return lax.rsqrt(jnp.maximum(deg, 1.0))


def _tc_pre_body(x_ref, w_ref, dg_ref, o_ref):
    dinv = _dinv_of(dg_ref)
    o_ref[...] = jnp.dot(x_ref[...], w_ref[...],
                         preferred_element_type=jnp.float32) * dinv


def _tc_pre(x_pad, W1, degp):
    grid = (N_PAD // BLK,)
    return pl.pallas_call(
        _tc_pre_body,
        grid=grid,
        in_specs=[
            pl.BlockSpec((BLK, 128), lambda i: (i, 0)),
            pl.BlockSpec((128, 128), lambda i: (0, 0)),
            pl.BlockSpec((2, BLK, 128), lambda i: (0, i, 0)),
        ],
        out_specs=pl.BlockSpec((BLK, 128), lambda i: (i, 0)),
        out_shape=jax.ShapeDtypeStruct((N_PAD, 128), jnp.float32),
    )(x_pad, W1, degp)


def _tc_mid_body(p_ref, dg_ref, b_ref, g_ref, be_ref, rm_ref, rv_ref, w_ref,
                 o_ref):
    dinv = _dinv_of(dg_ref)
    agg = (p_ref[0] + p_ref[1]) * dinv + b_ref[...]
    a = g_ref[...] * lax.rsqrt(rv_ref[...] + 1e-5)
    h = jnp.maximum((agg - rm_ref[...]) * a + be_ref[...], 0.0)
    o_ref[...] = jnp.dot(h, w_ref[...],
                         preferred_element_type=jnp.float32) * dinv


def _tc_mid(part, degp, b, g, be, rm, rv, W):
    grid = (N_PAD // BLK,)
    vspec = pl.BlockSpec((1, 128), lambda i: (0, 0))
    return pl.pallas_call(
        _tc_mid_body,
        grid=grid,
        in_specs=[
            pl.BlockSpec((2, BLK, 128), lambda i: (0, i, 0)),
            pl.BlockSpec((2, BLK, 128), lambda i: (0, i, 0)),
            vspec, vspec, vspec, vspec, vspec,
            pl.BlockSpec((128, 128), lambda i: (0, 0)),
        ],
        out_specs=pl.BlockSpec((BLK, 128), lambda i: (i, 0)),
        out_shape=jax.ShapeDtypeStruct((N_PAD, 128), jnp.float32),
    )(part, degp, b, g, be, rm, rv, W)


def _tc_head_body(p_ref, dg_ref, b_ref, g_ref, be_ref, rm_ref, rv_ref,
                  w1_ref, c1_ref, w2_ref, c2_ref, w3_ref, c3_ref, w4_ref,
                  c4_ref, o_ref):
    dinv = _dinv_of(dg_ref)
    agg = (p_ref[0] + p_ref[1]) * dinv + b_ref[...]
    a = g_ref[...] * lax.rsqrt(rv_ref[...] + 1e-5)
    h = jnp.maximum((agg - rm_ref[...]) * a + be_ref[...], 0.0)
    h = jnp.maximum(jnp.dot(h, w1_ref[...],
                            preferred_element_type=jnp.float32) + c1_ref[...],
                    0.0)
    h = jnp.maximum(jnp.dot(h, w2_ref[...],
                            preferred_element_type=jnp.float32) + c2_ref[...],
                    0.0)
    h = jnp.maximum(jnp.dot(h, w3_ref[...],
                            preferred_element_type=jnp.float32) + c3_ref[...],
                    0.0)
    o_ref[...] = jnp.dot(h, w4_ref[...],
                         preferred_element_type=jnp.float32) + c4_ref[...]


def _tc_head(part, degp, b, g, be, rm, rv, w1, c1, w2, c2, w3, c3, w4, c4):
    grid = (N_PAD // BLK,)

    def vs(d):
        return pl.BlockSpec((1, d), lambda i: (0, 0))

    def ws(a, bdim):
        return pl.BlockSpec((a, bdim), lambda i: (0, 0))

    return pl.pallas_call(
        _tc_head_body,
        grid=grid,
        in_specs=[
            pl.BlockSpec((2, BLK, 128), lambda i: (0, i, 0)),
            pl.BlockSpec((2, BLK, 128), lambda i: (0, i, 0)),
            vs(128), vs(128), vs(128), vs(128), vs(128),
            ws(128, 128), vs(128),
            ws(128, 64), vs(64),
            ws(64, 32), vs(32),
            ws(32, 16), vs(16),
        ],
        out_specs=pl.BlockSpec((BLK, 16), lambda i: (i, 0)),
        out_shape=jax.ShapeDtypeStruct((N_PAD, 16), jnp.float32),
    )(part, degp, b, g, be, rm, rv, w1, c1, w2, c2, w3, c3, w4, c4)


def kernel(x, W1, b1, g1, be1, rm1, rv1, W2, b2, g2, be2, rm2, rv2,
           W3, b3, g3, be3, rm3, rv3, lw1, lb1, lw2, lb2, lw3, lb3,
           lw4, lb4, edge_index):
    n = x.shape[0]
    e = edge_index.shape[1]
    loop = jnp.arange(n, dtype=jnp.int32)
    src = jnp.concatenate([edge_index[0].astype(jnp.int32), loop])
    dst = jnp.concatenate([edge_index[1].astype(jnp.int32), loop])
    e_tot = e + n
    steps = -(-e_tot // (NTILES * CHUNK))
    if steps % 2:
        steps += 1
    e_pad = steps * NTILES * CHUNK
    npad_e = e_pad - e_tot
    srcb = jnp.concatenate([src, jnp.zeros((npad_e,), jnp.int32)])
    dstb = jnp.concatenate(
        [dst, n + (jnp.arange(npad_e, dtype=jnp.int32) % (N_PAD - n))])
    dst3 = dstb.reshape(NTILES, steps, CHUNK)

    cap_h = 16 * S_HEAVY * CHUNK
    rest = e_tot - cap_h
    s_light = -(-rest // (16 * CHUNK))
    if s_light % 2:
        s_light += 1
    cap_l = 16 * s_light * CHUNK
    padl = cap_l - rest
    srcp = jnp.concatenate([src, jnp.zeros((padl,), jnp.int32)])
    dstp = jnp.concatenate(
        [dst, n + (jnp.arange(padl, dtype=jnp.int32) % (N_PAD - n))])
    sh = srcp[:cap_h].reshape(16, S_HEAVY, CHUNK)
    dh = dstp[:cap_h].reshape(16, S_HEAVY, CHUNK)
    sl = jnp.pad(srcp[cap_h:].reshape(16, s_light, CHUNK),
                 ((0, 0), (0, S_HEAVY - s_light), (0, 0)))
    dl = jnp.pad(dstp[cap_h:].reshape(16, s_light, CHUNK),
                 ((0, 0), (0, S_HEAVY - s_light), (0, 0)))
    if HEAVY_CORE == 0:
        src3 = jnp.concatenate([sh, sl])
        dst3s = jnp.concatenate([dh, dl])
        st0, st1 = S_HEAVY, s_light
    else:
        src3 = jnp.concatenate([sl, sh])
        dst3s = jnp.concatenate([dl, dh])
        st0, st1 = s_light, S_HEAVY

    x_pad = jnp.pad(x, ((0, N_PAD - n), (0, 0)))
    row = lambda v: v.reshape(1, -1)

    degp = _sc_degree(dst3)
    p1 = _tc_pre(x_pad, W1, degp)
    s1 = _sc_aggregate(p1, src3, dst3s, st0, st1)
    p2 = _tc_mid(s1, degp, row(b1), row(g1), row(be1), row(rm1), row(rv1), W2)
    s2 = _sc_aggregate(p2, src3, dst3s, st0, st1)
    p3 = _tc_mid(s2, degp, row(b2), row(g2), row(be2), row(rm2), row(rv2), W3)
    s3 = _sc_aggregate(p3, src3, dst3s, st0, st1)
    out = _tc_head(s3, degp, row(b3), row(g3), row(be3), row(rm3), row(rv3),
                   lw1.T, row(lb1), lw2.T, row(lb2), lw3.T, row(lb3),
                   lw4.T, row(lb4))
    return out[:n]


# speedup vs baseline: 1.0450x; 1.0450x over previous
import functools

import jax
import jax.numpy as jnp
from jax import lax
from jax.experimental import pallas as pl
from jax.experimental.pallas import tpu as pltpu
from jax.experimental.pallas import tpu_sc as plsc

N_PAD = 10240
NTILES = 32
CHUNK = 128
ROWS_PER_TILE = N_PAD // 16
BLK = 1024
S_HEAVY = 108
HEAVY_CORE = 0


def _sc_mesh():
    return plsc.VectorSubcoreMesh(core_axis_name="c", subcore_axis_name="s")


def _deg_body(dst_hbm, out_hbm, dst_v, buf, acc_sh, ssem):
    c = lax.axis_index("c")
    sid = lax.axis_index("s")
    wid = c * 16 + sid
    S = dst_v.shape[0]
    one16 = jnp.ones((16,), jnp.float32)
    zero16 = jnp.zeros((16,), jnp.float32)

    def zb(i, _):
        buf[i // 8, pl.ds((i % 8) * 16, 16)] = zero16
        return 0

    lax.fori_loop(0, CHUNK * 8, zb, 0)
    r0 = sid * ROWS_PER_TILE
    for j in range(ROWS_PER_TILE // CHUNK):
        pltpu.sync_copy(buf, acc_sh.at[pl.ds(r0 + j * CHUNK, CHUNK)])
    rem = ROWS_PER_TILE % CHUNK
    if rem:
        pltpu.sync_copy(
            buf.at[pl.ds(0, rem)],
            acc_sh.at[pl.ds(r0 + (ROWS_PER_TILE // CHUNK) * CHUNK, rem)])

    def ob(i, _):
        buf[i // 8, pl.ds((i % 8) * 16, 16)] = one16
        return 0

    lax.fori_loop(0, CHUNK * 8, ob, 0)
    pltpu.sync_copy(dst_hbm.at[wid], dst_v)
    plsc.subcore_barrier()

    def step(t, _):
        pltpu.async_copy(buf, acc_sh.at[dst_v.at[t]], ssem, add=True)
        return 0

    lax.fori_loop(0, S, step, 0)

    def drain(t, _):
        pltpu.make_async_copy(buf, acc_sh.at[dst_v.at[t]], ssem).wait()
        return 0

    lax.fori_loop(0, S, drain, 0)
    plsc.subcore_barrier()
    pltpu.sync_copy(acc_sh.at[pl.ds(r0, ROWS_PER_TILE)],
                    out_hbm.at[c, pl.ds(r0, ROWS_PER_TILE)])


def _sc_degree(dst3):
    S = dst3.shape[1]
    fn = pl.kernel(
        _deg_body,
        out_type=jax.ShapeDtypeStruct((2, N_PAD, 128), jnp.float32),
        mesh=_sc_mesh(),
        scratch_types=[
            pltpu.VMEM((S, CHUNK), jnp.int32),
            pltpu.VMEM((CHUNK, 128), jnp.float32),
            pltpu.VMEM_SHARED((N_PAD, 128), jnp.float32),
            pltpu.SemaphoreType.DMA,
        ],
    )
    return fn(dst3)


def _agg_body(p0, p1, h_hbm, src_hbm, dst_hbm, out_hbm, src_v, dstr, buf0,
              buf1, acc_sh, gs0, gs1, ss0, ss1, ds0, ds1):
    c = lax.axis_index("c")
    sid = lax.axis_index("s")
    wid = c * 16 + sid
    P = lax.convert_element_type(p0 + (p1 - p0) * c, jnp.int32)
    zero16 = jnp.zeros((16,), jnp.float32)

    def zb(i, _):
        buf0[i // 8, pl.ds((i % 8) * 16, 16)] = zero16
        return 0

    lax.fori_loop(0, CHUNK * 8, zb, 0)
    r0 = sid * ROWS_PER_TILE
    for j in range(ROWS_PER_TILE // CHUNK):
        pltpu.sync_copy(buf0, acc_sh.at[pl.ds(r0 + j * CHUNK, CHUNK)])
    pltpu.sync_copy(src_hbm.at[wid], src_v)
    plsc.subcore_barrier()

    pltpu.async_copy(dst_hbm.at[wid, 0], dstr.at[0], ds0)
    pltpu.async_copy(dst_hbm.at[wid, 1], dstr.at[1], ds1)
    pltpu.async_copy(h_hbm.at[src_v.at[0]], buf0, gs0)
    pltpu.async_copy(h_hbm.at[src_v.at[1]], buf1, gs1)

    def pair(p, _):
        t0 = 2 * p
        t1 = t0 + 1
        pltpu.make_async_copy(h_hbm.at[src_v.at[t0]], buf0, gs0).wait()
        pltpu.make_async_copy(dst_hbm.at[wid, t0], dstr.at[0], ds0).wait()
        pltpu.async_copy(buf0, acc_sh.at[dstr.at[0]], ss0, add=True)
        pltpu.make_async_copy(h_hbm.at[src_v.at[t1]], buf1, gs1).wait()
        pltpu.make_async_copy(dst_hbm.at[wid, t1], dstr.at[1], ds1).wait()
        pltpu.async_copy(buf1, acc_sh.at[dstr.at[1]], ss1, add=True)
        pltpu.make_async_copy(buf0, acc_sh.at[dstr.at[0]], ss0).wait()

        @pl.when(p + 1 < P)
        def _():
            pltpu.async_copy(dst_hbm.at[wid, t0 + 2], dstr.at[0], ds0)
            pltpu.async_copy(h_hbm.at[src_v.at[t0 + 2]], buf0, gs0)

        pltpu.make_async_copy(buf1, acc_sh.at[dstr.at[1]], ss1).wait()

        @pl.when(p + 1 < P)
        def _():
            pltpu.async_copy(dst_hbm.at[wid, t1 + 2], dstr.at[1], ds1)
            pltpu.async_copy(h_hbm.at[src_v.at[t1 + 2]], buf1, gs1)

        return 0

    lax.fori_loop(0, P, pair, 0)
    plsc.subcore_barrier()
    pltpu.sync_copy(acc_sh.at[pl.ds(r0, ROWS_PER_TILE)],
                    out_hbm.at[c, pl.ds(r0, ROWS_PER_TILE)])


def _sc_aggregate(h, src3, dst3, s0, s1):
    S = src3.shape[1]
    fn = pl.kernel(
        functools.partial(_agg_body, s0 // 2, s1 // 2),
        out_type=jax.ShapeDtypeStruct((2, N_PAD, 128), jnp.float32),
        mesh=_sc_mesh(),
        scratch_types=[
            pltpu.VMEM((S, CHUNK), jnp.int32),
            pltpu.VMEM((2, CHUNK), jnp.int32),
            pltpu.VMEM((CHUNK, 128), jnp.float32),
            pltpu.VMEM((CHUNK, 128), jnp.float32),
            pltpu.VMEM_SHARED((N_PAD, 128), jnp.float32),
            pltpu.SemaphoreType.DMA,
            pltpu.SemaphoreType.DMA,
            pltpu.SemaphoreType.DMA,
            pltpu.SemaphoreType.DMA,
            pltpu.SemaphoreType.DMA,
            pltpu.SemaphoreType.DMA,
        ],
    )
    return fn(h, src3, dst3)


def _dinv_of(dg_ref):
    deg = dg_ref[0, :, 0:1] + dg_ref[1, :, 0:1]
    return lax.rsqrt(jnp.maximum(deg, 1.0))


def _tc_pre_body(x_ref, w_ref, dg_ref, o_ref):
    dinv = _dinv_of(dg_ref)
    o_ref[...] = jnp.dot(x_ref[...], w_ref[...],
                         preferred_element_type=jnp.float32) * dinv


def _tc_pre(x_pad, W1, degp):
    grid = (N_PAD // BLK,)
    return pl.pallas_call(
        _tc_pre_body,
        grid=grid,
        in_specs=[
            pl.BlockSpec((BLK, 128), lambda i: (i, 0)),
            pl.BlockSpec((128, 128), lambda i: (0, 0)),
            pl.BlockSpec((2, BLK, 128), lambda i: (0, i, 0)),
        ],
        out_specs=pl.BlockSpec((BLK, 128), lambda i: (i, 0)),
        out_shape=jax.ShapeDtypeStruct((N_PAD, 128), jnp.float32),
    )(x_pad, W1, degp)


def _tc_mid_body(p_ref, dg_ref, b_ref, g_ref, be_ref, rm_ref, rv_ref, w_ref,
                 o_ref):
    dinv = _dinv_of(dg_ref)
    agg = (p_ref[0] + p_ref[1]) * dinv + b_ref[...]
    a = g_ref[...] * lax.rsqrt(rv_ref[...] + 1e-5)
    h = jnp.maximum((agg - rm_ref[...]) * a + be_ref[...], 0.0)
    o_ref[...] = jnp.dot(h, w_ref[...],
                         preferred_element_type=jnp.float32) * dinv


def _tc_mid(part, degp, b, g, be, rm, rv, W):
    grid = (N_PAD // BLK,)
    vspec = pl.BlockSpec((1, 128), lambda i: (0, 0))
    return pl.pallas_call(
        _tc_mid_body,
        grid=grid,
        in_specs=[
            pl.BlockSpec((2, BLK, 128), lambda i: (0, i, 0)),
            pl.BlockSpec((2, BLK, 128), lambda i: (0, i, 0)),
            vspec, vspec, vspec, vspec, vspec,
            pl.BlockSpec((128, 128), lambda i: (0, 0)),
        ],
        out_specs=pl.BlockSpec((BLK, 128), lambda i: (i, 0)),
        out_shape=jax.ShapeDtypeStruct((N_PAD, 128), jnp.float32),
    )(part, degp, b, g, be, rm, rv, W)


def _tc_head_body(p_ref, dg_ref, b_ref, g_ref, be_ref, rm_ref, rv_ref,
                  w1_ref, c1_ref, w2_ref, c2_ref, w3_ref, c3_ref, w4_ref,
                  c4_ref, o_ref):
    dinv = _dinv_of(dg_ref)
    agg = (p_ref[0] + p_ref[1]) * dinv + b_ref[...]
    a = g_ref[...] * lax.rsqrt(rv_ref[...] + 1e-5)
    h = jnp.maximum((agg - rm_ref[...]) * a + be_ref[...], 0.0)
    h = jnp.maximum(jnp.dot(h, w1_ref[...],
                            preferred_element_type=jnp.float32) + c1_ref[...],
                    0.0)
    h = jnp.maximum(jnp.dot(h, w2_ref[...],
                            preferred_element_type=jnp.float32) + c2_ref[...],
                    0.0)
    h = jnp.maximum(jnp.dot(h, w3_ref[...],
                            preferred_element_type=jnp.float32) + c3_ref[...],
                    0.0)
    o_ref[...] = jnp.dot(h, w4_ref[...],
                         preferred_element_type=jnp.float32) + c4_ref[...]


def _tc_head(part, degp, b, g, be, rm, rv, w1, c1, w2, c2, w3, c3, w4, c4):
    grid = (N_PAD // BLK,)

    def vs(d):
        return pl.BlockSpec((1, d), lambda i: (0, 0))

    def ws(a, bdim):
        return pl.BlockSpec((a, bdim), lambda i: (0, 0))

    return pl.pallas_call(
        _tc_head_body,
        grid=grid,
        in_specs=[
            pl.BlockSpec((2, BLK, 128), lambda i: (0, i, 0)),
            pl.BlockSpec((2, BLK, 128), lambda i: (0, i, 0)),
            vs(128), vs(128), vs(128), vs(128), vs(128),
            ws(128, 128), vs(128),
            ws(128, 64), vs(64),
            ws(64, 32), vs(32),
            ws(32, 16), vs(16),
        ],
        out_specs=pl.BlockSpec((BLK, 16), lambda i: (i, 0)),
        out_shape=jax.ShapeDtypeStruct((N_PAD, 16), jnp.float32),
    )(part, degp, b, g, be, rm, rv, w1, c1, w2, c2, w3, c3, w4, c4)


def kernel(x, W1, b1, g1, be1, rm1, rv1, W2, b2, g2, be2, rm2, rv2,
           W3, b3, g3, be3, rm3, rv3, lw1, lb1, lw2, lb2, lw3, lb3,
           lw4, lb4, edge_index):
    n = x.shape[0]
    e = edge_index.shape[1]
    loop = jnp.arange(n, dtype=jnp.int32)
    src = jnp.concatenate([edge_index[0].astype(jnp.int32), loop])
    dst = jnp.concatenate([edge_index[1].astype(jnp.int32), loop])
    e_tot = e + n
    steps = -(-e_tot // (NTILES * CHUNK))
    if steps % 2:
        steps += 1
    e_pad = steps * NTILES * CHUNK
    npad_e = e_pad - e_tot
    srcb = jnp.concatenate([src, jnp.zeros((npad_e,), jnp.int32)])
    dstb = jnp.concatenate(
        [dst, n + (jnp.arange(npad_e, dtype=jnp.int32) % (N_PAD - n))])
    dst3 = dstb.reshape(NTILES, steps, CHUNK)

    cap_h = 16 * S_HEAVY * CHUNK
    rest = e_tot - cap_h
    s_light = -(-rest // (16 * CHUNK))
    if s_light % 2:
        s_light += 1
    cap_l = 16 * s_light * CHUNK
    padl = cap_l - rest
    srcp = jnp.concatenate([src, jnp.zeros((padl,), jnp.int32)])
    dstp = jnp.concatenate(
        [dst, n + (jnp.arange(padl, dtype=jnp.int32) % (N_PAD - n))])
    sh = srcp[:cap_h].reshape(16, S_HEAVY, CHUNK)
    dh = dstp[:cap_h].reshape(16, S_HEAVY, CHUNK)
    sl = jnp.pad(srcp[cap_h:].reshape(16, s_light, CHUNK),
                 ((0, 0), (0, S_HEAVY - s_light), (0, 0)))
    dl = jnp.pad(dstp[cap_h:].reshape(16, s_light, CHUNK),
                 ((0, 0), (0, S_HEAVY - s_light), (0, 0)))
    if HEAVY_CORE == 0:
        src3 = jnp.concatenate([sh, sl])
        dst3s = jnp.concatenate([dh, dl])
        st0, st1 = S_HEAVY, s_light
    else:
        src3 = jnp.concatenate([sl, sh])
        dst3s = jnp.concatenate([dl, dh])
        st0, st1 = s_light, S_HEAVY

    x_pad = jnp.pad(x, ((0, N_PAD - n), (0, 0)))
    row = lambda v: v.reshape(1, -1)

    degp = _sc_degree(dst3)
    p1 = _tc_pre(x_pad, W1, degp)
    s1 = _sc_aggregate(p1, src3, dst3s, st0, st1)
    p2 = _tc_mid(s1, degp, row(b1), row(g1), row(be1), row(rm1), row(rv1), W2)
    s2 = _sc_aggregate(p2, src3, dst3s, st0, st1)
    p3 = _tc_mid(s2, degp, row(b2), row(g2), row(be2), row(rm2), row(rv2), W3)
    s3 = _sc_aggregate(p3, src3, dst3s, st0, st1)
    out = _tc_head(s3, degp, row(b3), row(g3), row(be3), row(rm3), row(rv3),
                   lw1.T, row(lb1), lw2.T, row(lb2), lw3.T, row(lb3),
                   lw4.T, row(lb4))
    return out[:n]
